# baseline (speedup 1.0000x reference)
import numpy as np
import jax
import jax.numpy as jnp
from jax import lax
from jax.experimental import pallas as pl
from jax.experimental.pallas import tpu as pltpu, tpu_sc as plsc

VOCAB = 100000
D = 128
B = 1024
L = 200
LANES = 16

NC = 2
NS = 16
NW = NC * NS

N = B * L
ROWS_W = N // NW
C = 128
G = ROWS_W // C
K = 2

PE_PAD = 208


def _sinusoidal_pe_padded():
    pos = np.arange(L, dtype=np.float32)[:, None]
    div = np.exp(np.arange(0, D, 2, dtype=np.float32) * -(np.log(10000.0) / D))
    pe = np.zeros((PE_PAD, D), dtype=np.float32)
    pe[:L, 0::2] = np.sin(pos * div)
    pe[:L, 1::2] = np.cos(pos * div)
    return pe


_PE = _sinusoidal_pe_padded()


def _body(tok_hbm, pe_hbm, seg_hbm, seq_hbm, segl_hbm, out_hbm,
          pe_v, idx_all, tok_b0, res_b0, tok_b1, res_b1, segl_v,
          sg0, sw0, ss0, sg1, sw1, ss1):
    cid = lax.axis_index("c")
    sid = lax.axis_index("s")
    wid = sid * NC + cid
    row0 = wid * ROWS_W

    pltpu.sync_copy(seq_hbm.at[wid], idx_all)
    pltpu.sync_copy(pe_hbm, pe_v)
    pltpu.sync_copy(seg_hbm, res_b0.at[pl.ds(0, 3)])
    seg_regs = [[res_b0[s, pl.ds(j * LANES, LANES)] for j in range(D // LANES)]
                for s in range(3)]

    slots = ((tok_b0, res_b0, sg0, sw0, ss0),
             (tok_b1, res_b1, sg1, sw1, ss1))

    def issue_fetch(g, s):
        tok_b, _, sg, _, ss = slots[s]
        pltpu.async_copy(tok_hbm.at[idx_all.at[g]], tok_b, sg)
        pltpu.async_copy(segl_hbm.at[wid, g], segl_v.at[s], ss)

    def wait_fetch(s):
        tok_b, _, sg, _, ss = slots[s]
        pltpu.make_async_copy(tok_hbm.at[pl.ds(0, C)], tok_b, sg).wait()
        pltpu.make_async_copy(segl_hbm.at[0, 0], segl_v.at[s], ss).wait()

    def add_chunk(t, s):
        tok_b, res_b = slots[s][0], slots[s][1]
        base = row0 + t * C

        def addgroup(i0, c2):
            sv = segl_v[s, pl.ds(i0 * LANES, LANES)]
            for r in range(LANES):
                i = i0 * LANES + r
                l = lax.rem(base + i, L)
                sgid = sv[r]
                is1 = sgid == 1
                is2 = sgid == 2
                for j in range(D // LANES):
                    sl = pl.ds(j * LANES, LANES)
                    segrow = jnp.where(is2, seg_regs[2][j],
                                       jnp.where(is1, seg_regs[1][j], seg_regs[0][j]))
                    res_b[i, sl] = tok_b[i, sl] + pe_v[l, sl] + segrow
            return c2

        lax.fori_loop(0, C // LANES, addgroup, 0)

    def issue_write(g, s):
        res_b, sw = slots[s][1], slots[s][3]
        pltpu.async_copy(res_b, out_hbm.at[pl.ds(row0 + g * C, C)], sw)

    def wait_write(s):
        res_b, sw = slots[s][1], slots[s][3]
        pltpu.make_async_copy(res_b, out_hbm.at[pl.ds(0, C)], sw).wait()

    for b in range(K):
        issue_fetch(b, b)

    def step(i, c2):
        for b in range(K):
            t = K * i + b
            wait_fetch(b)

            @pl.when(i >= 1)
            def _():
                wait_write(b)

            add_chunk(t, b)

            @pl.when(t + K < G)
            def _():
                issue_fetch(t + K, b)

            issue_write(t, b)
        return c2

    lax.fori_loop(0, G // K, step, 0)
    for b in range(K):
        wait_write(b)
    return


_SCRATCH = [
    pltpu.VMEM((PE_PAD, D), jnp.float32),
    pltpu.VMEM((G, C), jnp.int32),
    pltpu.VMEM((C, D), jnp.float32),
    pltpu.VMEM((C, D), jnp.float32),
    pltpu.VMEM((C, D), jnp.float32),
    pltpu.VMEM((C, D), jnp.float32),
    pltpu.VMEM((K, C), jnp.int32),
    pltpu.SemaphoreType.DMA,
    pltpu.SemaphoreType.DMA,
    pltpu.SemaphoreType.DMA,
    pltpu.SemaphoreType.DMA,
    pltpu.SemaphoreType.DMA,
    pltpu.SemaphoreType.DMA,
]

_sc_call = pl.kernel(
    _body,
    out_type=jax.ShapeDtypeStruct((N, D), jnp.float32),
    mesh=plsc.VectorSubcoreMesh(core_axis_name="c", subcore_axis_name="s"),
    scratch_types=_SCRATCH,
)


def kernel(sequence, segment_label, token_table, seg_table):
    seq = sequence.reshape(NW, G, C).astype(jnp.int32)
    segl = segment_label.reshape(NW, G, C).astype(jnp.int32)
    pe = jnp.asarray(_PE)
    out = _sc_call(token_table, pe, seg_table, seq, segl)
    return out.reshape(B, L, D)


# speedup vs baseline: 3.0184x; 3.0184x over previous
import numpy as np
import jax
import jax.numpy as jnp
from jax import lax
from jax.experimental import pallas as pl
from jax.experimental.pallas import tpu as pltpu, tpu_sc as plsc

VOCAB = 100000
D = 128
B = 1024
L = 200
LANES = 16

NC = 2
NS = 16
NW = NC * NS

N = B * L
ROWS_W = N // NW
C = 128
G = ROWS_W // C
K = 2
SPLIT = 2

LPT = 16
PE_PAD = NS * LPT
COMB_ROWS = PE_PAD * 3


def _sinusoidal_pe_padded():
    pos = np.arange(L, dtype=np.float32)[:, None]
    div = np.exp(np.arange(0, D, 2, dtype=np.float32) * -(np.log(10000.0) / D))
    pe = np.zeros((PE_PAD, D), dtype=np.float32)
    pe[:L, 0::2] = np.sin(pos * div)
    pe[:L, 1::2] = np.cos(pos * div)
    return pe


_PE = _sinusoidal_pe_padded()


def _body(tok_hbm, pe_hbm, seg_hbm, seq_hbm, segl_hbm, out_hbm,
          comb_sh, idx_all, idx2_all,
          tok_b0, pes_b0, res_b0, tok_b1, pes_b1, res_b1,
          sg0, sp0, sw0, sg1, sp1, sw1):
    cid = lax.axis_index("c")
    sid = lax.axis_index("s")
    wid = sid * NC + cid
    row0 = wid * ROWS_W

    pltpu.sync_copy(seq_hbm.at[wid], idx_all)
    for b in range(K):
        for h in range(SPLIT):
            pltpu.async_copy(
                tok_hbm.at[idx_all.at[b, pl.ds(h * (C // SPLIT), C // SPLIT)]],
                (tok_b0 if b == 0 else tok_b1).at[pl.ds(h * (C // SPLIT), C // SPLIT)],
                (sg0 if b == 0 else sg1))

    l_lo = sid * LPT
    pe_v, seg_v, comb_v = res_b0, res_b1, res_b0
    pltpu.sync_copy(pe_hbm.at[pl.ds(l_lo, LPT)], pe_v.at[pl.ds(64, LPT)])
    pltpu.sync_copy(seg_hbm, seg_v.at[pl.ds(0, 3)])
    for ll in range(LPT):
        for s in range(3):
            for j in range(D // LANES):
                sl = pl.ds(j * LANES, LANES)
                comb_v[ll * 3 + s, sl] = pe_v[ll + 64, sl] + seg_v[s, sl]
    pltpu.sync_copy(comb_v.at[pl.ds(0, LPT * 3)], comb_sh.at[pl.ds(l_lo * 3, LPT * 3)])
    plsc.subcore_barrier()

    pltpu.sync_copy(segl_hbm.at[wid], idx2_all)
    iota = lax.iota(jnp.int32, LANES)

    def mkidx(g, c2):
        for k in range(C // LANES):
            sl = pl.ds(k * LANES, LANES)
            flat = row0 + g * C + k * LANES + iota
            idx2_all[g, sl] = lax.rem(flat, L) * 3 + idx2_all[g, sl]
        return c2

    lax.fori_loop(0, G, mkidx, 0)

    slots = ((tok_b0, pes_b0, res_b0, sg0, sp0, sw0),
             (tok_b1, pes_b1, res_b1, sg1, sp1, sw1))

    def issue_gathers(g, s):
        tok_b, pes_b, _, sg, sp, _ = slots[s]
        for h in range(SPLIT):
            pltpu.async_copy(tok_hbm.at[idx_all.at[g, pl.ds(h * (C // SPLIT), C // SPLIT)]],
                             tok_b.at[pl.ds(h * (C // SPLIT), C // SPLIT)], sg)
        pltpu.async_copy(comb_sh.at[idx2_all.at[g]], pes_b, sp)

    def wait_gathers(s):
        tok_b, pes_b, _, sg, sp, _ = slots[s]
        pltpu.make_async_copy(tok_hbm.at[pl.ds(0, C)], tok_b, sg).wait()
        pltpu.make_async_copy(tok_hbm.at[pl.ds(0, C)], pes_b, sp).wait()

    H = C // 2

    def add_half(s, h):
        tok_b, pes_b, res_b = slots[s][0], slots[s][1], slots[s][2]

        def addrows(i, c2):
            for r in range(4):
                row = h * H + i * 4 + r
                for j in range(D // LANES):
                    sl = pl.ds(j * LANES, LANES)
                    res_b[row, sl] = tok_b[row, sl] + pes_b[row, sl]
            return c2

        lax.fori_loop(0, H // 4, addrows, 0)

    def issue_write_half(g, s, h):
        res_b, sw = slots[s][2], slots[s][5]
        pltpu.async_copy(res_b.at[pl.ds(h * H, H)],
                         out_hbm.at[pl.ds(row0 + g * C + h * H, H)], sw)

    def wait_write(s):
        res_b, sw = slots[s][2], slots[s][5]
        for h in range(2):
            pltpu.make_async_copy(res_b.at[pl.ds(h * H, H)],
                                  out_hbm.at[pl.ds(0, H)], sw).wait()

    for b in range(K):
        pltpu.async_copy(comb_sh.at[idx2_all.at[b]], slots[b][1], slots[b][4])

    def step(i, c2):
        for b in range(K):
            t = K * i + b
            wait_gathers(b)

            @pl.when(i >= 1)
            def _():
                wait_write(b)

            add_half(b, 0)
            issue_write_half(t, b, 0)
            add_half(b, 1)

            @pl.when(t + K < G)
            def _():
                issue_gathers(t + K, b)

            issue_write_half(t, b, 1)
        return c2

    lax.fori_loop(0, G // K, step, 0)
    for b in range(K):
        wait_write(b)
    return


_SCRATCH = [
    pltpu.VMEM_SHARED((COMB_ROWS, D), jnp.float32),
    pltpu.VMEM((G, C), jnp.int32),
    pltpu.VMEM((G, C), jnp.int32),
    pltpu.VMEM((C, D), jnp.float32),
    pltpu.VMEM((C, D), jnp.float32),
    pltpu.VMEM((C, D), jnp.float32),
    pltpu.VMEM((C, D), jnp.float32),
    pltpu.VMEM((C, D), jnp.float32),
    pltpu.VMEM((C, D), jnp.float32),
    pltpu.SemaphoreType.DMA,
    pltpu.SemaphoreType.DMA,
    pltpu.SemaphoreType.DMA,
    pltpu.SemaphoreType.DMA,
    pltpu.SemaphoreType.DMA,
    pltpu.SemaphoreType.DMA,
]

_sc_call = pl.kernel(
    _body,
    out_type=jax.ShapeDtypeStruct((N, D), jnp.float32),
    mesh=plsc.VectorSubcoreMesh(core_axis_name="c", subcore_axis_name="s"),
    scratch_types=_SCRATCH,
)


def kernel(sequence, segment_label, token_table, seg_table):
    seq = sequence.reshape(NW, G, C).astype(jnp.int32)
    segl = segment_label.reshape(NW, G, C).astype(jnp.int32)
    pe = jnp.asarray(_PE)
    out = _sc_call(token_table, pe, seg_table, seq, segl)
    return out.reshape(B, L, D)
